# baseline (speedup 1.0000x reference)
import jax
import jax.numpy as jnp
from jax.experimental import pallas as pl
from jax.experimental.pallas import tpu as pltpu

P = 16
DM = 384
CIN = 3
K = 1024
BETA = 0.25
PD = CIN * P * P * P

HB = 7
WN = 14
RT = HB * WN
NSLAB = CIN * P


def _vq_body(x_ref, we_ref, be_ref, cb_ref, wd_ref, bd_ref,
             y_ref, idx_ref, loss_ref):
    z = jnp.zeros((RT, DM), jnp.float32)
    for ct in range(NSLAB):
        c, tp = ct // P, ct % P
        a = x_ref[0, c, tp]
        a = a.reshape(HB, P, WN, P).transpose(0, 2, 1, 3).reshape(RT, P * P)
        z = z + jnp.dot(a, we_ref[pl.ds(ct * P * P, P * P), :],
                        preferred_element_type=jnp.float32)
    z = z + be_ref[...]
    cb = cb_ref[...]
    dot = jax.lax.dot_general(z, cb, (((1,), (1,)), ((), ())),
                              preferred_element_type=jnp.float32)
    znorm = jnp.sum(z * z, axis=1, keepdims=True)
    cnorm = jnp.sum(cb * cb, axis=1)[None, :]
    d2 = znorm - 2.0 * dot + cnorm
    dmin = jnp.min(d2, axis=1, keepdims=True)
    col = jax.lax.broadcasted_iota(jnp.int32, (RT, K), 1)
    idx = jnp.min(jnp.where(d2 <= dmin, col, K), axis=1)
    idx_ref[0, 0, :] = idx
    onehot = (col == idx[:, None]).astype(jnp.float32)
    zq = jnp.dot(onehot, cb, preferred_element_type=jnp.float32)
    diff = zq - z
    loss_ref[...] = jnp.sum(diff * diff).reshape(1, 1, 1)
    zqh = zq.astype(jnp.bfloat16)
    bd = bd_ref[...]
    for ct in range(NSLAB):
        c, tp = ct // P, ct % P
        yct = jnp.dot(zqh, wd_ref[:, pl.ds(ct * P * P, P * P)],
                      preferred_element_type=jnp.float32)
        yct = yct + bd[ct][None, :]
        yct = yct.reshape(HB, WN, P, P).transpose(0, 2, 1, 3)
        y_ref[0, c, tp] = yct.reshape(HB, P, WN * P)


def kernel(x, W_enc, b_enc, codebook, W_dec, b_dec):
    B, C, T, H, W = x.shape
    t, h, w = T // P, H // P, W // P
    N = t * h * w
    M = B * N
    G = M // RT

    x6 = x.reshape(B, C, T, h, P, W)
    wdh = W_dec.astype(jnp.bfloat16)
    bd2 = b_dec.reshape(NSLAB, P * P)

    y6, idx3, loss_parts = pl.pallas_call(
        _vq_body,
        grid=(B, t, h // HB),
        in_specs=[
            pl.BlockSpec((1, C, P, HB, P, W),
                         lambda b, ti, hh: (b, 0, ti, hh, 0, 0)),
            pl.BlockSpec((PD, DM), lambda b, ti, hh: (0, 0)),
            pl.BlockSpec((1, DM), lambda b, ti, hh: (0, 0)),
            pl.BlockSpec((K, DM), lambda b, ti, hh: (0, 0)),
            pl.BlockSpec((DM, PD), lambda b, ti, hh: (0, 0)),
            pl.BlockSpec((NSLAB, P * P), lambda b, ti, hh: (0, 0)),
        ],
        out_specs=[
            pl.BlockSpec((1, C, P, HB, P, W),
                         lambda b, ti, hh: (b, 0, ti, hh, 0, 0)),
            pl.BlockSpec((1, 1, RT),
                         lambda b, ti, hh: ((b * t + ti) * 2 + hh, 0, 0)),
            pl.BlockSpec((1, 1, 1),
                         lambda b, ti, hh: ((b * t + ti) * 2 + hh, 0, 0)),
        ],
        out_shape=[
            jax.ShapeDtypeStruct((B, C, T, h, P, W), jnp.float32),
            jax.ShapeDtypeStruct((G, 1, RT), jnp.int32),
            jax.ShapeDtypeStruct((G, 1, 1), jnp.float32),
        ],
        compiler_params=pltpu.CompilerParams(
            dimension_semantics=("arbitrary", "arbitrary", "arbitrary"),
            vmem_limit_bytes=60 * 1024 * 1024,
        ),
    )(x6, W_enc, b_enc.reshape(1, DM), codebook, wdh, bd2)

    loss = (1.0 + BETA) * jnp.sum(loss_parts) / (M * DM)
    encoding_indices = idx3.reshape(B, N)
    x_rec = y6.reshape(B, C, T, H, W)
    return x_rec, loss, encoding_indices
